# baseline (speedup 1.0000x reference)
import functools

import jax
import jax.numpy as jnp
from jax import lax
from jax.experimental import pallas as pl
from jax.experimental.pallas import tpu as pltpu
from jax.experimental.pallas import tpu_sc as plsc

NUM_CORES = 2
NUM_SUBCORES = 16
NUM_WORKERS = NUM_CORES * NUM_SUBCORES
LANES = 16

BATCH = 16384
EMBED_DIM = 64

CHUNK = 128


def _hole_body(head_hbm, rel_hbm, tail_hbm, etab_hbm, rtab_hbm, out_hbm,
               hraw, rraw, traw, hidx, ridx, tidx, hrows, trows, rrows,
               obuf, sem, *, rows_per_worker):
  wid = lax.axis_index("s") * NUM_CORES + lax.axis_index("c")
  base = wid * rows_per_worker
  nchunk = rows_per_worker // CHUNK

  pltpu.sync_copy(head_hbm.at[pl.ds(base, rows_per_worker)], hraw)
  pltpu.sync_copy(tail_hbm.at[pl.ds(base, rows_per_worker)], traw)
  pltpu.sync_copy(rel_hbm.at[pl.ds(base, rows_per_worker)], rraw)

  def half_body(i, carry):
    sl = pl.ds(i * LANES, LANES)
    hidx[sl] = lax.shift_right_logical(hraw[sl], 1)
    tidx[sl] = lax.shift_right_logical(traw[sl], 1)
    ridx[sl] = lax.shift_right_logical(rraw[sl], 1)
    return carry

  lax.fori_loop(0, rows_per_worker // LANES, half_body, 0, unroll=4)

  lanes = lax.iota(jnp.int32, LANES)

  def chunk_body(c, carry):
    cb = c * CHUNK
    sl = pl.ds(cb, CHUNK)
    ch = pltpu.make_async_copy(etab_hbm.at[hidx.at[sl]], hrows, sem)
    ct = pltpu.make_async_copy(etab_hbm.at[tidx.at[sl]], trows, sem)
    cr = pltpu.make_async_copy(rtab_hbm.at[ridx.at[sl]], rrows, sem)
    ch.start(); ct.start(); cr.start()
    ch.wait(); ct.wait(); cr.wait()

    def group_body(g, carry2):
      rows = lanes + g * LANES
      hoff = (hraw[pl.ds(cb + g * LANES, LANES)] & 1) * EMBED_DIM
      toff = (traw[pl.ds(cb + g * LANES, LANES)] & 1) * EMBED_DIM
      roff = (rraw[pl.ds(cb + g * LANES, LANES)] & 1) * EMBED_DIM

      def dot_body(d, acc):
        hv = plsc.load_gather(hrows, [rows, hoff + d])
        tv = plsc.load_gather(trows, [rows, toff + d])
        return acc + hv * tv

      corr = lax.fori_loop(0, EMBED_DIM, dot_body,
                           jnp.zeros((LANES,), jnp.float32), unroll=8)

      def out_body(d, carry3):
        rv = plsc.load_gather(rrows, [rows, roff + d])
        x = corr * rv
        obuf[d, pl.ds(g * LANES, LANES)] = 1.0 / (1.0 + jnp.exp(-x))
        return carry3

      lax.fori_loop(0, EMBED_DIM, out_body, 0, unroll=8)
      return carry2

    lax.fori_loop(0, CHUNK // LANES, group_body, 0)
    pltpu.sync_copy(obuf, out_hbm.at[:, pl.ds(base + cb, CHUNK)])
    return carry

  lax.fori_loop(0, nchunk, chunk_body, 0)


def _build(batch, interpret=False):
  rows_per_worker = batch // NUM_WORKERS
  mesh = plsc.VectorSubcoreMesh(core_axis_name="c", subcore_axis_name="s",
                                num_cores=NUM_CORES,
                                num_subcores=NUM_SUBCORES)
  return pl.kernel(
      functools.partial(_hole_body, rows_per_worker=rows_per_worker),
      out_type=jax.ShapeDtypeStruct((EMBED_DIM, batch), jnp.float32),
      mesh=mesh,
      scratch_types=[
          pltpu.VMEM((rows_per_worker,), jnp.int32),
          pltpu.VMEM((rows_per_worker,), jnp.int32),
          pltpu.VMEM((rows_per_worker,), jnp.int32),
          pltpu.VMEM((rows_per_worker,), jnp.int32),
          pltpu.VMEM((rows_per_worker,), jnp.int32),
          pltpu.VMEM((rows_per_worker,), jnp.int32),
          pltpu.VMEM((CHUNK, 2 * EMBED_DIM), jnp.float32),
          pltpu.VMEM((CHUNK, 2 * EMBED_DIM), jnp.float32),
          pltpu.VMEM((CHUNK, 2 * EMBED_DIM), jnp.float32),
          pltpu.VMEM((EMBED_DIM, CHUNK), jnp.float32),
          pltpu.SemaphoreType.DMA,
      ],
      compiler_params=pltpu.CompilerParams(needs_layout_passes=False),
      interpret=interpret,
  )


_hole = _build(BATCH)


def kernel(head, relation, tail, entity_table, relation_table):
  head = head.astype(jnp.int32)
  relation = relation.astype(jnp.int32)
  tail = tail.astype(jnp.int32)
  n_ent, dim = entity_table.shape
  n_rel, _ = relation_table.shape
  etab2 = entity_table.reshape(n_ent // 2, 2 * dim)
  rtab2 = relation_table.reshape(n_rel // 2, 2 * dim)
  out_t = _hole(head, relation, tail, etab2, rtab2)
  return out_t.T
